# baseline (speedup 1.0000x reference)
import functools

import jax
import jax.numpy as jnp
from jax.experimental import pallas as pl

N = 10000
E = 160000
C = 16
K = 5

_BE = 2048


def _msg_body(xj_ref, w_ref, logH_ref, out_ref):
    xj = xj_ref[...]
    w = w_ref[...]
    logH = logH_ref[...]
    m = jnp.max(xj, axis=-1, keepdims=True)
    u = xj - m
    t = u[:, :, None] + w[:, :, None] * logH[None, :, :]
    s = jnp.sum(jnp.exp(t), axis=1)
    out_ref[...] = jnp.log(s / jnp.sum(s, axis=-1, keepdims=True))


@functools.partial(jax.jit, static_argnames=())
def _messages(xj, w, logH):
    grid = (E + _BE - 1) // _BE
    return pl.pallas_call(
        _msg_body,
        grid=(grid,),
        in_specs=[
            pl.BlockSpec((_BE, C), lambda i: (i, 0)),
            pl.BlockSpec((_BE, 1), lambda i: (i, 0)),
            pl.BlockSpec((C, C), lambda i: (0, 0)),
        ],
        out_specs=pl.BlockSpec((_BE, C), lambda i: (i, 0)),
        out_shape=jax.ShapeDtypeStruct((E, C), jnp.float32),
    )(xj, w, logH)


def kernel(x, edge_index, edge_weight, edge_rv, W1, b1, W2, b2, param):
    h = jnp.maximum(x @ W1 + b1, 0.0) @ W2 + b2
    logH = jax.nn.log_sigmoid(param + param.T)
    src = edge_index[0]
    dst = edge_index[1]
    w = edge_weight[:, None]

    raw = h
    prev = None
    for _ in range(K):
        xj = raw[src]
        if prev is not None:
            xj = xj - prev[edge_rv]
        msg = _messages(xj, w, logH)
        prev = msg
        agg = jax.ops.segment_sum(msg, dst, num_segments=N)
        raw = h + agg
    return raw - jax.scipy.special.logsumexp(raw, axis=-1, keepdims=True)


# speedup vs baseline: 1.6545x; 1.6545x over previous
import functools

import jax
import jax.numpy as jnp
from jax.experimental import pallas as pl

N = 10000
E = 160000
C = 16
K = 5

_BE = 2048


def _msg_body(xj_ref, w_ref, logH_ref, out_ref):
    xjT = xj_ref[...].T
    wT = w_ref[...].T
    logH = logH_ref[...]
    m = jnp.max(xjT, axis=0, keepdims=True)
    u = xjT - m
    rows = []
    for c2 in range(C):
        t = u + logH[:, c2:c2 + 1] * wT
        rows.append(jnp.sum(jnp.exp(t), axis=0, keepdims=True))
    sig = jnp.concatenate(rows, axis=0)
    msgT = jnp.log(sig / jnp.sum(sig, axis=0, keepdims=True))
    out_ref[...] = msgT.T


@functools.partial(jax.jit, static_argnames=())
def _messages(xj, w, logH):
    grid = (E + _BE - 1) // _BE
    return pl.pallas_call(
        _msg_body,
        grid=(grid,),
        in_specs=[
            pl.BlockSpec((_BE, C), lambda i: (i, 0)),
            pl.BlockSpec((_BE, 1), lambda i: (i, 0)),
            pl.BlockSpec((C, C), lambda i: (0, 0)),
        ],
        out_specs=pl.BlockSpec((_BE, C), lambda i: (i, 0)),
        out_shape=jax.ShapeDtypeStruct((E, C), jnp.float32),
    )(xj, w, logH)


def kernel(x, edge_index, edge_weight, edge_rv, W1, b1, W2, b2, param):
    h = jnp.maximum(x @ W1 + b1, 0.0) @ W2 + b2
    logH = jax.nn.log_sigmoid(param + param.T)
    src = edge_index[0]
    dst = edge_index[1]
    w = edge_weight[:, None]

    raw = h
    prev = None
    for _ in range(K):
        xj = raw[src]
        if prev is not None:
            xj = xj - prev[edge_rv]
        msg = _messages(xj, w, logH)
        prev = msg
        agg = jax.ops.segment_sum(msg, dst, num_segments=N)
        raw = h + agg
    return raw - jax.scipy.special.logsumexp(raw, axis=-1, keepdims=True)
